# baseline (speedup 1.0000x reference)
import jax
import jax.numpy as jnp
from jax.experimental import pallas as pl


def _body(a_ref, o_ref):
    col = a_ref[...].reshape(a_ref.shape[1], 1)
    o_ref[...] = jnp.broadcast_to(col, o_ref.shape)


def kernel(x, action):
    B = x.shape[0]
    A = action.shape[0]
    a2 = action.reshape(1, A)
    g = 4
    wide = pl.pallas_call(
        _body,
        grid=(g,),
        in_specs=[pl.BlockSpec((1, A), lambda i: (0, 0))],
        out_specs=pl.BlockSpec((A, B // g), lambda i: (0, i)),
        out_shape=jax.ShapeDtypeStruct((A, B), jnp.float32),
    )(a2)
    return wide.T


# speedup vs baseline: 1.3726x; 1.3726x over previous
import jax
import jax.numpy as jnp
from jax.experimental import pallas as pl
from jax.experimental.pallas import tpu as pltpu

_K = 4


def _body(a_ref, o_hbm, buf, sems):
    A, B = o_hbm.shape
    c = B // _K
    col = a_ref[...].reshape(A, 1)
    copies = []
    for k in range(_K):
        buf[:, pl.ds(k * c, c)] = jnp.broadcast_to(col, (A, c))
        cp = pltpu.make_async_copy(
            buf.at[:, pl.ds(k * c, c)],
            o_hbm.at[:, pl.ds(k * c, c)],
            sems.at[k],
        )
        cp.start()
        copies.append(cp)
    for cp in copies:
        cp.wait()


def kernel(x, action):
    B = x.shape[0]
    A = action.shape[0]
    a2 = action.reshape(1, A)
    wide = pl.pallas_call(
        _body,
        in_specs=[pl.BlockSpec((1, A), lambda: (0, 0))],
        out_specs=pl.BlockSpec(memory_space=pl.ANY),
        out_shape=jax.ShapeDtypeStruct((A, B), jnp.float32),
        scratch_shapes=[
            pltpu.VMEM((A, B), jnp.float32),
            pltpu.SemaphoreType.DMA((_K,)),
        ],
    )(a2)
    return wide.T
